# baseline (speedup 1.0000x reference)
import functools

import jax
import jax.numpy as jnp
from jax import lax
from jax.experimental import pallas as pl
from jax.experimental.pallas import tpu as pltpu
from jax.experimental.pallas import tpu_sc as plsc

_NUM_WORKERS = 32
_MAIN = 24
_UNIT = 8


def kernel(indices, embedding_weight):
    batch, tokens = indices.shape
    vocab, dim = embedding_weight.shape
    rows = batch * tokens
    tail = rows - _NUM_WORKERS * _MAIN
    subdim = dim * tail // (_NUM_WORKERS * _UNIT)

    idx_flat = indices.reshape(-1).astype(jnp.int32)
    mesh = plsc.VectorSubcoreMesh(core_axis_name="c", subcore_axis_name="s")

    @functools.partial(
        pl.kernel,
        mesh=mesh,
        out_type=jax.ShapeDtypeStruct((rows, dim), jnp.float32),
        scratch_types=[
            pltpu.VMEM((32,), jnp.int32),
            pltpu.VMEM((8, dim), jnp.float32),
            pltpu.VMEM((16, dim), jnp.float32),
            pltpu.SemaphoreType.DMA,
            pltpu.SemaphoreType.DMA,
        ],
    )
    def gather_kernel(table_hbm, idx_hbm, out_hbm, idx_v, buf_a, buf_b, gsem, wsem):
        wid = lax.axis_index("s") * 2 + lax.axis_index("c")
        off = wid * _MAIN
        qrow = _NUM_WORKERS * _MAIN + (wid // 8) * _UNIT
        qcol = (wid % 8) * subdim

        s0 = pltpu.async_copy(
            idx_hbm.at[pl.ds(off, _MAIN)], idx_v.at[pl.ds(0, _MAIN)], gsem
        )
        s1 = pltpu.async_copy(
            idx_hbm.at[pl.ds(qrow, _UNIT)], idx_v.at[pl.ds(_MAIN, _UNIT)], gsem
        )
        s0.wait()
        s1.wait()

        def gather(i0, n, buf):
            return pltpu.async_copy(
                table_hbm.at[idx_v.at[pl.ds(i0, n)]], buf, gsem
            )

        g0 = gather(0, 8, buf_a)
        g1 = gather(8, 16, buf_b)
        g0.wait()
        w0 = pltpu.async_copy(buf_a, out_hbm.at[pl.ds(off, 8)], wsem)
        g1.wait()
        w1 = pltpu.async_copy(buf_b, out_hbm.at[pl.ds(off + 8, 16)], wsem)
        w0.wait()
        g2 = gather(_MAIN, _UNIT, buf_a)
        g2.wait()
        w2 = pltpu.async_copy(
            buf_a.at[:, pl.ds(qcol, subdim)],
            out_hbm.at[pl.ds(qrow, _UNIT), pl.ds(qcol, subdim)],
            wsem,
        )
        w1.wait()
        w2.wait()

    out = gather_kernel(embedding_weight, idx_flat)
    return out.reshape(batch, tokens, dim)


# speedup vs baseline: 1.0478x; 1.0478x over previous
import functools

import jax
import jax.numpy as jnp
from jax import lax
from jax.experimental import pallas as pl
from jax.experimental.pallas import tpu as pltpu
from jax.experimental.pallas import tpu_sc as plsc

_NUM_WORKERS = 32
_LIGHT = 28


def kernel(indices, embedding_weight):
    batch, tokens = indices.shape
    vocab, dim = embedding_weight.shape
    rows = batch * tokens
    assert _LIGHT * 24 + (_NUM_WORKERS - _LIGHT) * 32 == rows

    idx_flat = indices.reshape(-1).astype(jnp.int32)
    mesh = plsc.VectorSubcoreMesh(core_axis_name="c", subcore_axis_name="s")

    @functools.partial(
        pl.kernel,
        mesh=mesh,
        out_type=jax.ShapeDtypeStruct((rows, dim), jnp.float32),
        scratch_types=[
            pltpu.VMEM((32,), jnp.int32),
            pltpu.VMEM((24, dim), jnp.float32),
            pltpu.SemaphoreType.DMA,
            pltpu.SemaphoreType.DMA,
        ],
    )
    def gather_kernel(table_hbm, idx_hbm, out_hbm, idx_v, buf, gsem, wsem):
        wid = lax.axis_index("s") * 2 + lax.axis_index("c")
        heavy = wid >= _LIGHT
        off = jnp.where(heavy, _LIGHT * 24 + (wid - _LIGHT) * 32, wid * 24)

        pltpu.sync_copy(idx_hbm.at[pl.ds(off, 32)], idx_v)

        g0 = pltpu.async_copy(
            table_hbm.at[idx_v.at[pl.ds(0, 24)]], buf, gsem
        )
        g0.wait()
        w0 = pltpu.async_copy(buf, out_hbm.at[pl.ds(off, 24)], wsem)
        w0.wait()

        @pl.when(heavy)
        def _():
            g1 = pltpu.async_copy(
                table_hbm.at[idx_v.at[pl.ds(24, 8)]], buf.at[pl.ds(0, 8)], gsem
            )
            g1.wait()
            pltpu.async_copy(
                buf.at[pl.ds(0, 8)], out_hbm.at[pl.ds(off + 24, 8)], wsem
            ).wait()

    out = gather_kernel(embedding_weight, idx_flat)
    return out.reshape(batch, tokens, dim)
